# Initial kernel scaffold; baseline (speedup 1.0000x reference)
import jax
import jax.numpy as jnp
from jax.experimental import pallas as pl


def kernel(user_indices, item_indices, user_table, item_table, W, b):
    raise NotImplementedError("write your pallas kernel here")


# speedup vs baseline: 1.4306x; 1.4306x over previous
import functools

import jax
import jax.numpy as jnp
from jax import lax
from jax.experimental import pallas as pl
from jax.experimental.pallas import tpu as pltpu
from jax.experimental.pallas import tpu_sc as plsc


@functools.lru_cache(maxsize=None)
def _make_sc_gather(B, D, NC, NS, CH=128):
    NW = NC * NS
    b_per_w = B // NW
    n_ch = b_per_w // CH
    mesh = plsc.VectorSubcoreMesh(core_axis_name="c", subcore_axis_name="s")

    @functools.partial(
        pl.kernel,
        mesh=mesh,
        out_type=[
            jax.ShapeDtypeStruct((B, D), jnp.float32),
            jax.ShapeDtypeStruct((B, D), jnp.float32),
        ],
        scratch_types=[
            pltpu.VMEM((n_ch, CH), jnp.int32),
            pltpu.VMEM((b_per_w, D), jnp.float32),
            pltpu.SemaphoreType.DMA,
        ],
    )
    def gather2(uidx_hbm, iidx_hbm, ut_hbm, it_hbm, out_u, out_i,
                idx_v, rows_v, sem):
        wid = lax.axis_index("s") * NC + lax.axis_index("c")
        base = wid * b_per_w
        for idx_hbm, tbl, out in ((uidx_hbm, ut_hbm, out_u),
                                  (iidx_hbm, it_hbm, out_i)):
            pltpu.sync_copy(idx_hbm.at[pl.ds(wid * n_ch, n_ch)], idx_v)
            cps = [
                pltpu.async_copy(tbl.at[idx_v.at[j]],
                                 rows_v.at[pl.ds(j * CH, CH)], sem)
                for j in range(n_ch)
            ]
            for c in cps:
                c.wait()
            pltpu.sync_copy(rows_v, out.at[pl.ds(base, b_per_w)])

    return gather2


def _dense_body(ue_ref, ie_ref, w_ref, b_ref, out_ref):
    w = w_ref[...]
    bb = b_ref[...]
    u = jnp.maximum(
        jnp.dot(ue_ref[...], w, preferred_element_type=jnp.float32,
                precision=lax.Precision.HIGHEST) + bb, 0.0)
    v = jnp.maximum(
        jnp.dot(ie_ref[...], w, preferred_element_type=jnp.float32,
                precision=lax.Precision.HIGHEST) + bb, 0.0)
    dot = jnp.sum(u * v, axis=1)
    un = jnp.maximum(jnp.sqrt(jnp.sum(u * u, axis=1)), 1e-8)
    vn = jnp.maximum(jnp.sqrt(jnp.sum(v * v, axis=1)), 1e-8)
    out_ref[...] = dot / (un * vn)


def kernel(user_indices, item_indices, user_table, item_table, W, b):
    B = user_indices.shape[0]
    D = user_table.shape[1]
    N = W.shape[1]
    NP = (N + 127) // 128 * 128
    CH = 128

    info = plsc.get_sparse_core_info()
    NC, NS = info.num_cores, info.num_subcores

    uidx = user_indices.astype(jnp.int32).reshape(B // CH, CH)
    iidx = item_indices.astype(jnp.int32).reshape(B // CH, CH)

    ue, ie = _make_sc_gather(B, D, NC, NS, CH)(
        uidx, iidx, user_table, item_table)

    Wp = jnp.pad(W, ((0, 0), (0, NP - N)))
    bp = jnp.pad(b, (0, NP - N)).reshape(1, NP)

    BM = 2048
    scores = pl.pallas_call(
        _dense_body,
        grid=(B // BM,),
        in_specs=[
            pl.BlockSpec((BM, D), lambda i: (i, 0)),
            pl.BlockSpec((BM, D), lambda i: (i, 0)),
            pl.BlockSpec((D, NP), lambda i: (0, 0)),
            pl.BlockSpec((1, NP), lambda i: (0, 0)),
        ],
        out_specs=pl.BlockSpec((BM,), lambda i: (i,)),
        out_shape=jax.ShapeDtypeStruct((B,), jnp.float32),
    )(ue, ie, Wp, bp)
    return scores


# speedup vs baseline: 2.2449x; 1.5693x over previous
import functools

import jax
import jax.numpy as jnp
from jax import lax
from jax.experimental import pallas as pl
from jax.experimental.pallas import tpu as pltpu
from jax.experimental.pallas import tpu_sc as plsc


@functools.lru_cache(maxsize=None)
def _make_sc_gather(B, D, NC, NS, CH=128):
    NW = NC * NS
    b_per_w = B // NW
    n_ch = b_per_w // CH
    mesh = plsc.VectorSubcoreMesh(core_axis_name="c", subcore_axis_name="s")

    @functools.partial(
        pl.kernel,
        mesh=mesh,
        out_type=[
            jax.ShapeDtypeStruct((B, D), jnp.float32),
            jax.ShapeDtypeStruct((B, D), jnp.float32),
        ],
        scratch_types=[
            pltpu.VMEM((n_ch, CH), jnp.int32),
            pltpu.VMEM((b_per_w, D), jnp.float32),
            pltpu.SemaphoreType.DMA,
        ],
    )
    def gather2(uidx_hbm, iidx_hbm, ut_hbm, it_hbm, out_u, out_i,
                idx_v, rows_v, sem):
        wid = lax.axis_index("s") * NC + lax.axis_index("c")
        base = wid * b_per_w
        for idx_hbm, tbl, out in ((uidx_hbm, ut_hbm, out_u),
                                  (iidx_hbm, it_hbm, out_i)):
            pltpu.sync_copy(idx_hbm.at[pl.ds(wid * n_ch, n_ch)], idx_v)
            cps = [
                pltpu.async_copy(tbl.at[idx_v.at[j]],
                                 rows_v.at[pl.ds(j * CH, CH)], sem)
                for j in range(n_ch)
            ]
            for c in cps:
                c.wait()
            pltpu.sync_copy(rows_v, out.at[pl.ds(base, b_per_w)])

    return gather2


def _dense_body(ue_ref, ie_ref, w_ref, b_ref, out_ref):
    w = w_ref[...]
    bb = b_ref[...]
    u = jnp.maximum(
        jnp.dot(ue_ref[...], w, preferred_element_type=jnp.float32) + bb, 0.0)
    v = jnp.maximum(
        jnp.dot(ie_ref[...], w, preferred_element_type=jnp.float32) + bb, 0.0)
    dot = jnp.sum(u * v, axis=1)
    un = jnp.maximum(jnp.sqrt(jnp.sum(u * u, axis=1)), 1e-8)
    vn = jnp.maximum(jnp.sqrt(jnp.sum(v * v, axis=1)), 1e-8)
    out_ref[...] = dot / (un * vn)


def kernel(user_indices, item_indices, user_table, item_table, W, b):
    B = user_indices.shape[0]
    D = user_table.shape[1]
    N = W.shape[1]
    NP = (N + 127) // 128 * 128
    CH = 128

    info = plsc.get_sparse_core_info()
    NC, NS = info.num_cores, info.num_subcores

    uidx = user_indices.astype(jnp.int32).reshape(B // CH, CH)
    iidx = item_indices.astype(jnp.int32).reshape(B // CH, CH)

    ue, ie = _make_sc_gather(B, D, NC, NS, CH)(
        uidx, iidx, user_table, item_table)

    Wp = jnp.pad(W, ((0, 0), (0, NP - N)))
    bp = jnp.pad(b, (0, NP - N)).reshape(1, NP)

    BM = 2048
    scores = pl.pallas_call(
        _dense_body,
        grid=(B // BM,),
        in_specs=[
            pl.BlockSpec((BM, D), lambda i: (i, 0)),
            pl.BlockSpec((BM, D), lambda i: (i, 0)),
            pl.BlockSpec((D, NP), lambda i: (0, 0)),
            pl.BlockSpec((1, NP), lambda i: (0, 0)),
        ],
        out_specs=pl.BlockSpec((BM,), lambda i: (i,)),
        out_shape=jax.ShapeDtypeStruct((B,), jnp.float32),
    )(ue, ie, Wp, bp)
    return scores


# speedup vs baseline: 2.2652x; 1.0090x over previous
import functools

import jax
import jax.numpy as jnp
from jax import lax
from jax.experimental import pallas as pl
from jax.experimental.pallas import tpu as pltpu
from jax.experimental.pallas import tpu_sc as plsc


@functools.lru_cache(maxsize=None)
def _make_sc_gather(B, D, NC, NS, CH=128):
    NW = NC * NS
    b_per_w = B // NW
    n_ch = b_per_w // CH
    mesh = plsc.VectorSubcoreMesh(core_axis_name="c", subcore_axis_name="s")

    @functools.partial(
        pl.kernel,
        mesh=mesh,
        out_type=[
            jax.ShapeDtypeStruct((B, D), jnp.float32),
            jax.ShapeDtypeStruct((B, D), jnp.float32),
        ],
        scratch_types=[
            pltpu.VMEM((n_ch, CH), jnp.int32),
            pltpu.VMEM((b_per_w, D), jnp.float32),
            pltpu.SemaphoreType.DMA,
        ],
    )
    def gather2(uidx_hbm, iidx_hbm, ut_hbm, it_hbm, out_u, out_i,
                idx_v, rows_v, sem):
        wid = lax.axis_index("s") * NC + lax.axis_index("c")
        base = wid * b_per_w
        for idx_hbm, tbl, out in ((uidx_hbm, ut_hbm, out_u),
                                  (iidx_hbm, it_hbm, out_i)):
            pltpu.sync_copy(idx_hbm.at[pl.ds(wid * n_ch, n_ch)], idx_v)
            cps = [
                pltpu.async_copy(tbl.at[idx_v.at[j]],
                                 rows_v.at[pl.ds(j * CH, CH)], sem)
                for j in range(n_ch)
            ]
            for c in cps:
                c.wait()
            pltpu.sync_copy(rows_v, out.at[pl.ds(base, b_per_w)])

    return gather2


def _dense_body(ue_ref, ie_ref, w_ref, b_ref, out_ref):
    w = w_ref[...].astype(jnp.bfloat16)
    bb = b_ref[...]
    u = jnp.maximum(
        jnp.dot(ue_ref[...].astype(jnp.bfloat16), w,
                preferred_element_type=jnp.float32) + bb, 0.0)
    v = jnp.maximum(
        jnp.dot(ie_ref[...].astype(jnp.bfloat16), w,
                preferred_element_type=jnp.float32) + bb, 0.0)
    dot = jnp.sum(u * v, axis=1)
    un = jnp.maximum(jnp.sqrt(jnp.sum(u * u, axis=1)), 1e-8)
    vn = jnp.maximum(jnp.sqrt(jnp.sum(v * v, axis=1)), 1e-8)
    out_ref[...] = dot / (un * vn)


def kernel(user_indices, item_indices, user_table, item_table, W, b):
    B = user_indices.shape[0]
    D = user_table.shape[1]
    N = W.shape[1]
    NP = (N + 127) // 128 * 128
    CH = 128

    info = plsc.get_sparse_core_info()
    NC, NS = info.num_cores, info.num_subcores

    uidx = user_indices.astype(jnp.int32).reshape(B // CH, CH)
    iidx = item_indices.astype(jnp.int32).reshape(B // CH, CH)

    ue, ie = _make_sc_gather(B, D, NC, NS, CH)(
        uidx, iidx, user_table, item_table)

    Wp = jnp.pad(W, ((0, 0), (0, NP - N)))
    bp = jnp.pad(b, (0, NP - N)).reshape(1, NP)

    BM = 2048
    scores = pl.pallas_call(
        _dense_body,
        grid=(B // BM,),
        in_specs=[
            pl.BlockSpec((BM, D), lambda i: (i, 0)),
            pl.BlockSpec((BM, D), lambda i: (i, 0)),
            pl.BlockSpec((D, NP), lambda i: (0, 0)),
            pl.BlockSpec((1, NP), lambda i: (0, 0)),
        ],
        out_specs=pl.BlockSpec((BM,), lambda i: (i,)),
        out_shape=jax.ShapeDtypeStruct((B,), jnp.float32),
    )(ue, ie, Wp, bp)
    return scores


# speedup vs baseline: 2.4587x; 1.0854x over previous
import functools

import jax
import jax.numpy as jnp
from jax import lax
from jax.experimental import pallas as pl
from jax.experimental.pallas import tpu as pltpu
from jax.experimental.pallas import tpu_sc as plsc


@functools.lru_cache(maxsize=None)
def _make_sc_gather(B, D, NC, NS, CH=128):
    NW = NC * NS
    b_per_w = B // NW
    n_ch = b_per_w // CH
    mesh = plsc.VectorSubcoreMesh(core_axis_name="c", subcore_axis_name="s")

    @functools.partial(
        pl.kernel,
        mesh=mesh,
        out_type=[
            jax.ShapeDtypeStruct((B, D), jnp.float32),
            jax.ShapeDtypeStruct((B, D), jnp.float32),
        ],
        scratch_types=[
            pltpu.VMEM((n_ch, CH), jnp.int32),
            pltpu.VMEM((b_per_w, D), jnp.float32),
            pltpu.SemaphoreType.DMA,
        ],
    )
    def gather2(uidx_hbm, iidx_hbm, ut_hbm, it_hbm, out_u, out_i,
                idx_v, rows_v, sem):
        wid = lax.axis_index("s") * NC + lax.axis_index("c")
        base = wid * b_per_w
        for idx_hbm, tbl, out in ((uidx_hbm, ut_hbm, out_u),
                                  (iidx_hbm, it_hbm, out_i)):
            pltpu.sync_copy(idx_hbm.at[pl.ds(wid * n_ch, n_ch)], idx_v)
            cps = [
                pltpu.async_copy(tbl.at[idx_v.at[j]],
                                 rows_v.at[pl.ds(j * CH, CH)], sem)
                for j in range(n_ch)
            ]
            for c in cps:
                c.wait()
            pltpu.sync_copy(rows_v, out.at[pl.ds(base, b_per_w)])

    return gather2


def _dense_body(ue_ref, ie_ref, w_ref, b_ref, out_ref):
    w = w_ref[...].astype(jnp.bfloat16)
    bb = b_ref[...]
    u = jnp.maximum(
        jnp.dot(ue_ref[...].astype(jnp.bfloat16), w,
                preferred_element_type=jnp.float32) + bb, 0.0)
    v = jnp.maximum(
        jnp.dot(ie_ref[...].astype(jnp.bfloat16), w,
                preferred_element_type=jnp.float32) + bb, 0.0)
    dots = jnp.sum(u * v, axis=1, keepdims=True)
    un = jnp.maximum(jnp.sqrt(jnp.sum(u * u, axis=1, keepdims=True)), 1e-8)
    vn = jnp.maximum(jnp.sqrt(jnp.sum(v * v, axis=1, keepdims=True)), 1e-8)
    out_ref[...] = dots / (un * vn)


def kernel(user_indices, item_indices, user_table, item_table, W, b):
    B = user_indices.shape[0]
    D = user_table.shape[1]
    N = W.shape[1]
    NP = (N + 127) // 128 * 128
    CH = 128

    info = plsc.get_sparse_core_info()
    NC, NS = info.num_cores, info.num_subcores

    uidx = user_indices.astype(jnp.int32).reshape(B // CH, CH)
    iidx = item_indices.astype(jnp.int32).reshape(B // CH, CH)

    ue, ie = _make_sc_gather(B, D, NC, NS, CH)(
        uidx, iidx, user_table, item_table)

    Wp = jnp.pad(W, ((0, 0), (0, NP - N)))
    bp = jnp.pad(b, (0, NP - N)).reshape(1, NP)

    BM = 2048
    scores = pl.pallas_call(
        _dense_body,
        grid=(B // BM,),
        in_specs=[
            pl.BlockSpec((BM, D), lambda i: (i, 0)),
            pl.BlockSpec((BM, D), lambda i: (i, 0)),
            pl.BlockSpec((D, NP), lambda i: (0, 0)),
            pl.BlockSpec((1, NP), lambda i: (0, 0)),
        ],
        out_specs=pl.BlockSpec((BM, 1), lambda i: (i, 0)),
        out_shape=jax.ShapeDtypeStruct((B, 1), jnp.float32),
    )(ue, ie, Wp, bp)
    return scores.reshape(B)
